# baseline (speedup 1.0000x reference)
import functools

import jax
import jax.numpy as jnp
from jax import lax
from jax.experimental import pallas as pl
from jax.experimental.pallas import tpu as pltpu
from jax.experimental.pallas import tpu_sc as plsc

N = 262144
D = 258
H = 256
O = 5
E = 8

B = 512
C = 2 * N + E * B
NB = C // B
TG = 1024
TP = 256
N5H = (N // 2) * 5
M5 = N5H + 2048

NC = 2
NS = 16
NW = NC * NS
CH = 256
BC = 5504

def _sc_mesh():
    return plsc.VectorSubcoreMesh(core_axis_name="c", subcore_axis_name="s")


def _gate_block(x_ref, Wg_ref, bg_ref,
                xm_ref, xtp_ref, i1_ref, i2_ref, g1_ref, g2_ref, cnt_ref,
                cacc):
    pid = pl.program_id(0)

    @pl.when(pid == 0)
    def _():
        cacc[...] = jnp.zeros_like(cacc)

    x = x_ref[...]
    logits = jnp.dot(x, Wg_ref[...], preferred_element_type=jnp.float32)
    logits = logits + bg_ref[...][None, :]
    i1 = jnp.argmax(logits, axis=-1)
    m1 = jnp.max(logits, axis=-1)
    eidx = lax.broadcasted_iota(jnp.int32, (TG, E), 1)
    masked = jnp.where(eidx == i1[:, None].astype(jnp.int32), -jnp.inf, logits)
    i2 = jnp.argmax(masked, axis=-1)
    m2 = jnp.max(masked, axis=-1)
    t = jnp.exp(m2 - m1)
    g1 = 1.0 / (1.0 + t)
    g2 = t * g1

    xm_ref[...] = x[:, :256]
    tb = x[:, 256:258]
    fb = lax.bitcast_convert_type(tb, jnp.int32)
    r = (fb + 0x7FFF + ((fb >> 16) & 1)) >> 16
    xtp_ref[...] = (r[:, 0] << 16) | (r[:, 1] & 0xFFFF)

    i1 = i1.astype(jnp.int32)
    i2 = i2.astype(jnp.int32)
    i1_ref[...] = i1
    i2_ref[...] = i2
    tid = pid * TG + lax.broadcasted_iota(jnp.int32, (TG,), 0)
    g1_ref[...] = tid * 4096 + jnp.round(g1 * 4095.0).astype(jnp.int32)
    g2_ref[...] = tid * 4096 + jnp.round(g2 * 4095.0).astype(jnp.int32)
    oh = ((eidx == i1[:, None]).astype(jnp.float32)
          + (eidx == i2[:, None]).astype(jnp.float32))
    cacc[...] = cacc[...] + jnp.sum(oh, axis=0, keepdims=True)
    cnt_ref[...] = cacc[...].astype(jnp.int32)


def _gate(x, Wg, bg):
    grid = (N // TG,)
    return pl.pallas_call(
        _gate_block,
        grid=grid,
        in_specs=[
            pl.BlockSpec((TG, D), lambda i: (i, 0)),
            pl.BlockSpec(Wg.shape, lambda i: (0, 0)),
            pl.BlockSpec(bg.shape, lambda i: (0,)),
        ],
        out_specs=[
            pl.BlockSpec((TG, 256), lambda i: (i, 0)),
            pl.BlockSpec((TG,), lambda i: (i,)),
            pl.BlockSpec((TG,), lambda i: (i,)),
            pl.BlockSpec((TG,), lambda i: (i,)),
            pl.BlockSpec((TG,), lambda i: (i,)),
            pl.BlockSpec((TG,), lambda i: (i,)),
            pl.BlockSpec((1, E), lambda i: (0, 0)),
        ],
        out_shape=[
            jax.ShapeDtypeStruct((N, 256), jnp.float32),
            jax.ShapeDtypeStruct((N,), jnp.int32),
            jax.ShapeDtypeStruct((N,), jnp.int32),
            jax.ShapeDtypeStruct((N,), jnp.int32),
            jax.ShapeDtypeStruct((N,), jnp.int32),
            jax.ShapeDtypeStruct((N,), jnp.int32),
            jax.ShapeDtypeStruct((1, E), jnp.int32),
        ],
        scratch_shapes=[pltpu.VMEM((1, E), jnp.float32)],
    )(x, Wg, bg)


def _pos_block(i1_ref, i2_ref, off_ref, ltri_ref, pos1_ref, pos2_ref, base):
    pid = pl.program_id(0)

    @pl.when(pid == 0)
    def _():
        base[...] = jnp.zeros_like(base)

    i1 = i1_ref[...]
    i2 = i2_ref[...]
    eidx = lax.broadcasted_iota(jnp.int32, (TP, E), 1)
    oh1 = (eidx == i1[:, None]).astype(jnp.float32)
    oh2 = (eidx == i2[:, None]).astype(jnp.float32)
    ohc = oh1 + oh2
    rk = jnp.dot(ltri_ref[...], ohc, preferred_element_type=jnp.float32)
    tot = off_ref[...].astype(jnp.float32) + base[...] + rk
    pos1_ref[...] = jnp.sum(oh1 * tot, axis=1).astype(jnp.int32)
    pos2_ref[...] = jnp.sum(oh2 * tot, axis=1).astype(jnp.int32)
    base[...] = base[...] + jnp.sum(ohc, axis=0, keepdims=True)


def _positions(i1, i2, off, ltri):
    grid = (N // TP,)
    return pl.pallas_call(
        _pos_block,
        grid=grid,
        in_specs=[
            pl.BlockSpec((TP,), lambda i: (i,)),
            pl.BlockSpec((TP,), lambda i: (i,)),
            pl.BlockSpec((1, E), lambda i: (0, 0)),
            pl.BlockSpec((TP, TP), lambda i: (0, 0)),
        ],
        out_specs=[
            pl.BlockSpec((TP,), lambda i: (i,)),
            pl.BlockSpec((TP,), lambda i: (i,)),
        ],
        out_shape=[
            jax.ShapeDtypeStruct((N,), jnp.int32),
            jax.ShapeDtypeStruct((N,), jnp.int32),
        ],
        scratch_shapes=[pltpu.VMEM((1, E), jnp.float32)],
    )(i1, i2, off, ltri)


def _dispatch(xm, gp1, gp2, pos1, pos2, sg_init):
  k = functools.partial(
    pl.kernel, mesh=_sc_mesh(),
    out_type=(
        jax.ShapeDtypeStruct((C, 256), jnp.float32),
        jax.ShapeDtypeStruct((NC * C,), jnp.int32),
    ),
    scratch_types=[
        pltpu.VMEM((CH, 256), jnp.float32),
        pltpu.VMEM((CH,), jnp.int32),
        pltpu.VMEM((CH,), jnp.int32),
        pltpu.VMEM((CH,), jnp.int32),
        pltpu.VMEM((CH,), jnp.int32),
        pltpu.VMEM_SHARED((C,), jnp.int32),
        pltpu.SemaphoreType.DMA,
    ],
  )(_dispatch_body)
  return k(xm, gp1, gp2, pos1, pos2, sg_init)


def _dispatch_body(xm, gp1, gp2, pos1, pos2, sg_init,
              xs_o, sg_o,
              rows_v, p1_v, p2_v, gp1_v, gp2_v,
              sg_s, sem):
    cid = lax.axis_index("c")
    sid = lax.axis_index("s")
    wid = sid * NC + cid
    cs = C // NS
    pltpu.sync_copy(sg_init.at[pl.ds(sid * cs, cs)], sg_s.at[pl.ds(sid * cs, cs)])
    plsc.subcore_barrier()

    tpw = N // NW
    base0 = wid * tpw

    def chunk(c, carry):
        b0 = base0 + c * CH
        pltpu.sync_copy(xm.at[pl.ds(b0, CH)], rows_v)
        pltpu.sync_copy(pos1.at[pl.ds(b0, CH)], p1_v)
        pltpu.sync_copy(pos2.at[pl.ds(b0, CH)], p2_v)
        pltpu.sync_copy(gp1.at[pl.ds(b0, CH)], gp1_v)
        pltpu.sync_copy(gp2.at[pl.ds(b0, CH)], gp2_v)
        pltpu.async_copy(rows_v, xs_o.at[p1_v], sem).wait()
        pltpu.async_copy(rows_v, xs_o.at[p2_v], sem).wait()
        pltpu.sync_copy(gp1_v, sg_s.at[p1_v])
        pltpu.sync_copy(gp2_v, sg_s.at[p2_v])
        return carry

    lax.fori_loop(0, tpw // CH, chunk, 0)
    plsc.subcore_barrier()
    pltpu.sync_copy(sg_s.at[pl.ds(sid * cs, cs)],
                    sg_o.at[pl.ds(cid * C + sid * cs, cs)])


def _dispatch_tail(xtp, pos1, pos2, zi):
  k = functools.partial(
    pl.kernel, mesh=_sc_mesh(),
    out_type=jax.ShapeDtypeStruct((NC * C,), jnp.int32),
    scratch_types=[
        pltpu.VMEM((CH,), jnp.int32),
        pltpu.VMEM((CH,), jnp.int32),
        pltpu.VMEM((CH,), jnp.int32),
        pltpu.VMEM_SHARED((C,), jnp.int32),
    ],
  )(_dispatch_tail_body)
  return k(xtp, pos1, pos2, zi)


def _dispatch_tail_body(xtp, pos1, pos2, zi, us_o,
                        p1_v, p2_v, tp_v, us_s):
    cid = lax.axis_index("c")
    sid = lax.axis_index("s")
    wid = sid * NC + cid
    cs = C // NS
    pltpu.sync_copy(zi.at[pl.ds(sid * cs, cs)], us_s.at[pl.ds(sid * cs, cs)])
    plsc.subcore_barrier()

    tpw = N // NW
    base0 = wid * tpw

    def chunk(c, carry):
        b0 = base0 + c * CH
        pltpu.sync_copy(pos1.at[pl.ds(b0, CH)], p1_v)
        pltpu.sync_copy(pos2.at[pl.ds(b0, CH)], p2_v)
        pltpu.sync_copy(xtp.at[pl.ds(b0, CH)], tp_v)
        pltpu.sync_copy(tp_v, us_s.at[p1_v])
        pltpu.sync_copy(tp_v, us_s.at[p2_v])
        return carry

    lax.fori_loop(0, tpw // CH, chunk, 0)
    plsc.subcore_barrier()
    pltpu.sync_copy(us_s.at[pl.ds(sid * cs, cs)],
                    us_o.at[pl.ds(cid * C + sid * cs, cs)])


def _mlp_block(be_ref, xs_ref, ut0_ref, ut1_ref, sg0_ref, sg1_ref,
               W1_ref, b1_ref, W2_ref, b2_ref, W3_ref, b3_ref,
               W4_ref, b4_ref, out_ref):
    xb = xs_ref[...].astype(jnp.bfloat16)
    W1b = W1_ref[0]
    h = jnp.dot(xb, W1b[:256], preferred_element_type=jnp.float32)
    ut = ut0_ref[0, 0] + ut1_ref[0, 0]
    ua = lax.bitcast_convert_type(ut & jnp.int32(-65536), jnp.float32)
    ub = lax.bitcast_convert_type(ut << 16, jnp.float32)
    u2 = jnp.concatenate([ua[:, None], ub[:, None]], axis=1).astype(jnp.bfloat16)
    h = h + jnp.dot(u2, W1b[256:258], preferred_element_type=jnp.float32)
    h = jnp.maximum(h + b1_ref[0], 0.0).astype(jnp.bfloat16)
    h = jnp.maximum(
        jnp.dot(h, W2_ref[0], preferred_element_type=jnp.float32)
        + b2_ref[0], 0.0).astype(jnp.bfloat16)
    h = jnp.maximum(
        jnp.dot(h, W3_ref[0], preferred_element_type=jnp.float32)
        + b3_ref[0], 0.0).astype(jnp.bfloat16)
    y = (jnp.dot(h, W4_ref[0], preferred_element_type=jnp.float32)
         + b4_ref[0])
    sp = jnp.minimum(sg0_ref[0, 0], sg1_ref[0, 0])
    g = (sp & 4095).astype(jnp.float32) * (1.0 / 4095.0)
    out_ref[...] = (y * g[:, None]).T


def _grouped_mlp(be, xs, ut0, ut1, sg0, sg1, W1, b1, W2, b2, W3, b3, W4p, b4p):
    grid_spec = pltpu.PrefetchScalarGridSpec(
        num_scalar_prefetch=1,
        grid=(NB,),
        in_specs=[
            pl.BlockSpec((B, 256), lambda i, be_: (i, 0)),
            pl.BlockSpec((1, 1, B), lambda i, be_: (i, 0, 0)),
            pl.BlockSpec((1, 1, B), lambda i, be_: (i, 0, 0)),
            pl.BlockSpec((1, 1, B), lambda i, be_: (i, 0, 0)),
            pl.BlockSpec((1, 1, B), lambda i, be_: (i, 0, 0)),
            pl.BlockSpec((1, D, 256), lambda i, be_: (be_[i], 0, 0)),
            pl.BlockSpec((1, 1, 256), lambda i, be_: (be_[i], 0, 0)),
            pl.BlockSpec((1, 256, 256), lambda i, be_: (be_[i], 0, 0)),
            pl.BlockSpec((1, 1, 256), lambda i, be_: (be_[i], 0, 0)),
            pl.BlockSpec((1, 256, 256), lambda i, be_: (be_[i], 0, 0)),
            pl.BlockSpec((1, 1, 256), lambda i, be_: (be_[i], 0, 0)),
            pl.BlockSpec((1, 256, 8), lambda i, be_: (be_[i], 0, 0)),
            pl.BlockSpec((1, 1, 8), lambda i, be_: (be_[i], 0, 0)),
        ],
        out_specs=pl.BlockSpec((8, B), lambda i, be_: (0, i)),
    )
    return pl.pallas_call(
        _mlp_block,
        grid_spec=grid_spec,
        out_shape=jax.ShapeDtypeStruct((8, C), jnp.float32),
    )(be, xs, ut0, ut1, sg0, sg1, W1, b1, W2, b2, W3, b3, W4p, b4p)


def _combine(ysw, sg, accz):
  k = functools.partial(
    pl.kernel, mesh=_sc_mesh(),
    out_type=jax.ShapeDtypeStruct((NC * M5,), jnp.float32),
    scratch_types=[
        pltpu.VMEM((BC,), jnp.int32),
        pltpu.VMEM((BC,), jnp.int32),
        pltpu.VMEM((BC,), jnp.int32),
        pltpu.VMEM((BC,), jnp.int32),
        pltpu.VMEM((BC,), jnp.float32),
        pltpu.VMEM_SHARED((M5,), jnp.float32),
    ],
  )(_combine_body)
  return k(ysw, sg, accz)


def _combine_body(ysw, sg, accz, out,
             t0_v, t1_v, tok5_v, idx_v, val_v, acc_s):
    cid = lax.axis_index("c")
    sid = lax.axis_index("s")
    ms = M5 // NS
    pltpu.sync_copy(accz.at[pl.ds(sid * ms, ms)], acc_s.at[pl.ds(sid * ms, ms)])
    plsc.subcore_barrier()

    sls = C // NS
    s0 = sid * sls
    lo = cid * N5H

    def chunk(c, carry):
        b0 = s0 + c * BC
        pltpu.sync_copy(sg.at[pl.ds(b0, BC)], t0_v)
        pltpu.sync_copy(sg.at[pl.ds(C + b0, BC)], t1_v)

        def mb(j, cc):
            sl = pl.ds(j * 16, 16)
            sp = jnp.minimum(t0_v[sl], t1_v[sl])
            loc = (sp >> 12) * 5 - lo
            ok = (loc >= 0) & (loc < N5H)
            tok5_v[sl] = jnp.where(ok, loc, N5H)
            return cc

        lax.fori_loop(0, BC // 16, mb, 0)
        for o in range(5):
            pltpu.sync_copy(ysw.at[pl.ds(o * C + b0, BC)], val_v)

            def ib(j, cc):
                sl = pl.ds(j * 16, 16)
                idx_v[sl] = tok5_v[sl] + o
                return cc

            lax.fori_loop(0, BC // 16, ib, 0)
            pltpu.sync_copy(val_v, acc_s.at[idx_v], add=True)
        return carry

    lax.fori_loop(0, sls // BC, chunk, 0)
    plsc.subcore_barrier()
    pltpu.sync_copy(acc_s.at[pl.ds(sid * ms, ms)],
                    out.at[pl.ds(cid * M5 + sid * ms, ms)])


def kernel(x, Wg, bg, W1, b1, W2, b2, W3, b3, W4, b4):
    xm, xtp, i1, i2, gp1, gp2, cnt = _gate(x, Wg, bg)

    counts = cnt[0]
    padded = ((counts + (B - 1)) // B) * B
    off = jnp.concatenate([jnp.zeros((1,), jnp.int32),
                           jnp.cumsum(padded)[:-1].astype(jnp.int32)])
    blk_off = off // B
    be = (jnp.sum((jnp.arange(NB, dtype=jnp.int32)[:, None]
                   >= blk_off[None, :]).astype(jnp.int32), axis=1) - 1)

    ltri = jnp.tril(jnp.ones((TP, TP), jnp.float32), -1)
    pos1, pos2 = _positions(i1, i2, off[None, :], ltri)

    sg_init = jnp.full((C,), N * 4096, jnp.int32)
    zi = jnp.zeros((C,), jnp.int32)
    xs, sg = _dispatch(xm, gp1, gp2, pos1, pos2, sg_init)
    us = _dispatch_tail(xtp, pos1, pos2, zi)

    W1b = W1.astype(jnp.bfloat16)
    W2b = W2.astype(jnp.bfloat16)
    W3b = W3.astype(jnp.bfloat16)
    W4p = jnp.pad(W4, ((0, 0), (0, 0), (0, 3))).astype(jnp.bfloat16)
    b4p = jnp.pad(b4, ((0, 0), (0, 3)))
    yswT = _grouped_mlp(be, xs,
                        us[:C].reshape(NB, 1, B), us[C:].reshape(NB, 1, B),
                        sg[:C].reshape(NB, 1, B), sg[C:].reshape(NB, 1, B),
                        W1b, b1[:, None], W2b, b2[:, None],
                        W3b, b3[:, None], W4p, b4p[:, None])

    accz = jnp.zeros((M5,), jnp.float32)
    acc = _combine(yswT.reshape(8 * C), sg, accz)
    out = jnp.concatenate([acc[:N5H], acc[M5:M5 + N5H]])
    return out.reshape(N, O)


# speedup vs baseline: 1.7788x; 1.7788x over previous
import functools

import jax
import jax.numpy as jnp
from jax.experimental import pallas as pl

N = 262144
D = 258
H = 256
O = 5
E = 8

T = 2048


def _moe_block(x_ref, Wg_ref, bg_ref, W1_ref, b1_ref, W2_ref, b2_ref,
               W3_ref, b3_ref, W4_ref, b4_ref, out_ref):
    x = x_ref[...]
    logits = jnp.dot(x, Wg_ref[...], preferred_element_type=jnp.float32)
    logits = logits + bg_ref[...][None, :]

    i1 = jnp.argmax(logits, axis=-1)
    m1 = jnp.max(logits, axis=-1)
    eidx = jax.lax.broadcasted_iota(jnp.int32, (x.shape[0], E), 1)
    masked = jnp.where(eidx == i1[:, None], -jnp.inf, logits)
    i2 = jnp.argmax(masked, axis=-1)
    m2 = jnp.max(masked, axis=-1)
    t = jnp.exp(m2 - m1)
    g1 = 1.0 / (1.0 + t)
    g2 = t * g1

    xb = x.astype(jnp.bfloat16)
    acc = jnp.zeros((x.shape[0], O), dtype=jnp.float32)
    for e in range(E):
        h = jnp.maximum(
            jnp.dot(xb, W1_ref[e], preferred_element_type=jnp.float32)
            + b1_ref[e][None, :], 0.0).astype(jnp.bfloat16)
        h = jnp.maximum(
            jnp.dot(h, W2_ref[e], preferred_element_type=jnp.float32)
            + b2_ref[e][None, :], 0.0).astype(jnp.bfloat16)
        h = jnp.maximum(
            jnp.dot(h, W3_ref[e], preferred_element_type=jnp.float32)
            + b3_ref[e][None, :], 0.0).astype(jnp.bfloat16)
        y = (jnp.dot(h, W4_ref[e], preferred_element_type=jnp.float32)
             + b4_ref[e][None, :])
        w = g1 * (i1 == e) + g2 * (i2 == e)
        acc = acc + w[:, None] * y
    out_ref[...] = acc


def kernel(x, Wg, bg, W1, b1, W2, b2, W3, b3, W4, b4):
    W1 = W1.astype(jnp.bfloat16)
    W2 = W2.astype(jnp.bfloat16)
    W3 = W3.astype(jnp.bfloat16)
    W4 = W4.astype(jnp.bfloat16)
    n = x.shape[0]
    grid = (n // T,)
    full = lambda a: pl.BlockSpec(a.shape, lambda i: (0,) * a.ndim)
    return pl.pallas_call(
        _moe_block,
        grid=grid,
        in_specs=[
            pl.BlockSpec((T, D), lambda i: (i, 0)),
            full(Wg), full(bg),
            full(W1), full(b1), full(W2), full(b2),
            full(W3), full(b3), full(W4), full(b4),
        ],
        out_specs=pl.BlockSpec((T, O), lambda i: (i, 0)),
        out_shape=jax.ShapeDtypeStruct((n, O), jnp.float32),
    )(x, Wg, bg, W1, b1, W2, b2, W3, b3, W4, b4)


# speedup vs baseline: 1.8665x; 1.0493x over previous
import functools

import jax
import jax.numpy as jnp
from jax.experimental import pallas as pl

N = 262144
D = 258
H = 256
O = 5
E = 8

T = 4096


def _moe_block(x_ref, Wg_ref, bg_ref, W1_ref, b1_ref, W2_ref, b2_ref,
               W3_ref, b3_ref, W4_ref, b4_ref, out_ref):
    x = x_ref[...]
    logits = jnp.dot(x, Wg_ref[...], preferred_element_type=jnp.float32)
    logits = logits + bg_ref[...][None, :]

    i1 = jnp.argmax(logits, axis=-1)
    m1 = jnp.max(logits, axis=-1)
    eidx = jax.lax.broadcasted_iota(jnp.int32, (x.shape[0], E), 1)
    masked = jnp.where(eidx == i1[:, None], -jnp.inf, logits)
    i2 = jnp.argmax(masked, axis=-1)
    m2 = jnp.max(masked, axis=-1)
    t = jnp.exp(m2 - m1)
    g1 = 1.0 / (1.0 + t)
    g2 = t * g1

    xb = x.astype(jnp.bfloat16)
    acc = jnp.zeros((x.shape[0], O), dtype=jnp.float32)
    for e in range(E):
        h = jnp.maximum(
            jnp.dot(xb, W1_ref[e], preferred_element_type=jnp.float32)
            + b1_ref[e][None, :], 0.0).astype(jnp.bfloat16)
        h = jnp.maximum(
            jnp.dot(h, W2_ref[e], preferred_element_type=jnp.float32)
            + b2_ref[e][None, :], 0.0).astype(jnp.bfloat16)
        h = jnp.maximum(
            jnp.dot(h, W3_ref[e], preferred_element_type=jnp.float32)
            + b3_ref[e][None, :], 0.0).astype(jnp.bfloat16)
        y = (jnp.dot(h, W4_ref[e], preferred_element_type=jnp.float32)
             + b4_ref[e][None, :])
        w = g1 * (i1 == e) + g2 * (i2 == e)
        acc = acc + w[:, None] * y
    out_ref[...] = acc


def kernel(x, Wg, bg, W1, b1, W2, b2, W3, b3, W4, b4):
    W1 = W1.astype(jnp.bfloat16)
    W2 = W2.astype(jnp.bfloat16)
    W3 = W3.astype(jnp.bfloat16)
    W4 = W4.astype(jnp.bfloat16)
    n = x.shape[0]
    grid = (n // T,)
    full = lambda a: pl.BlockSpec(a.shape, lambda i: (0,) * a.ndim)
    return pl.pallas_call(
        _moe_block,
        grid=grid,
        in_specs=[
            pl.BlockSpec((T, D), lambda i: (i, 0)),
            full(Wg), full(bg),
            full(W1), full(b1), full(W2), full(b2),
            full(W3), full(b3), full(W4), full(b4),
        ],
        out_specs=pl.BlockSpec((T, O), lambda i: (i, 0)),
        out_shape=jax.ShapeDtypeStruct((n, O), jnp.float32),
    )(x, Wg, bg, W1, b1, W2, b2, W3, b3, W4, b4)
